# Initial kernel scaffold; baseline (speedup 1.0000x reference)
import jax
import jax.numpy as jnp
from jax.experimental import pallas as pl


def kernel(logits, target):
    raise NotImplementedError("write your pallas kernel here")


# speedup vs baseline: 2.7061x; 2.7061x over previous
import functools

import jax
import jax.numpy as jnp
from jax.experimental import pallas as pl
from jax.experimental.pallas import tpu as pltpu

_C = 1000
_B = 16384
_S = 0.1
_COEF_B = _S / (_C - 1)
_COEF_A = 1.0 - _S - _COEF_B

_BLOCK_ROWS = 512


def _loss_body(x_ref, t_ref, out_ref):
    i = pl.program_id(0)
    x = x_ref[...]
    t = t_ref[...]
    m = jnp.max(x, axis=1, keepdims=True)
    e = jnp.exp(x - m)
    s = jnp.sum(e, axis=1, keepdims=True)
    lse = m + jnp.log(s)
    rowsum = jnp.sum(x, axis=1, keepdims=True)
    cols = jax.lax.broadcasted_iota(jnp.int32, x.shape, 1)
    tgt = jnp.sum(jnp.where(cols == t, x, 0.0), axis=1, keepdims=True)
    part = jnp.sum(lse - _COEF_A * tgt - _COEF_B * rowsum)

    @pl.when(i == 0)
    def _init():
        out_ref[0, 0] = part

    @pl.when(i != 0)
    def _acc():
        out_ref[0, 0] += part


@functools.partial(jax.jit, static_argnames=("interpret",))
def _loss(logits, target, interpret=False):
    t2d = target.reshape(_B, 1)
    grid = _B // _BLOCK_ROWS
    total = pl.pallas_call(
        _loss_body,
        grid=(grid,),
        in_specs=[
            pl.BlockSpec((_BLOCK_ROWS, _C), lambda i: (i, 0)),
            pl.BlockSpec((_BLOCK_ROWS, 1), lambda i: (i, 0)),
        ],
        out_specs=pl.BlockSpec(memory_space=pltpu.SMEM),
        out_shape=jax.ShapeDtypeStruct((1, 1), jnp.float32),
        interpret=interpret,
    )(logits, t2d)
    return total[0, 0] * (1.0 / _B)


def kernel(logits, target):
    return _loss(logits, target)
